# baseline (speedup 1.0000x reference)
import functools

import jax
import jax.numpy as jnp
from jax import lax
from jax.experimental import pallas as pl
from jax.experimental.pallas import tpu as pltpu
from jax.experimental.pallas import tpu_sc as plsc


def kernel(x, seq_emb_weight, pos_encoding):
    B, S = x.shape
    V, D = seq_emb_weight.shape
    N = B * S

    info = plsc.get_sparse_core_info()
    NC, NS, L = info.num_cores, info.num_subcores, info.num_lanes
    NW = NC * NS

    SPS = 4
    R = SPS * S
    NG = R // 128
    TAIL = R - NG * 128
    seq_per_w = B // NW
    n_slots = seq_per_w // SPS
    n_groups = n_slots // 2

    pos = pos_encoding[:S]

    mesh = plsc.VectorSubcoreMesh(core_axis_name="c", subcore_axis_name="s",
                                  num_cores=NC)

    @functools.partial(
        pl.kernel,
        mesh=mesh,
        out_type=jax.ShapeDtypeStruct((B, S, D), jnp.float32),
        compiler_params=pltpu.CompilerParams(use_tc_tiling_on_sc=False),
        scratch_types=[
            pltpu.VMEM((2, R), jnp.int32),
            pltpu.VMEM((2, R, D), jnp.float32),
            pltpu.VMEM((S, D), jnp.float32),
            pltpu.SemaphoreType.DMA((2,)),
            pltpu.SemaphoreType.DMA((2,)),
            pltpu.SemaphoreType.DMA((2,)),
        ],
    )
    def emb_pos_kernel(table_hbm, idx_hbm, pos_hbm, out_hbm,
                       idx_v, rows_v, pos_v, sem_idx, sem_g, sem_out):
        wid = lax.axis_index("s") * NC + lax.axis_index("c")
        wseq0 = wid * seq_per_w

        pltpu.sync_copy(pos_hbm, pos_v)

        def idx_copy(c, b):
            fb = (wseq0 + SPS * c) * S
            return pltpu.make_async_copy(
                idx_hbm.at[pl.ds(fb, R)], idx_v.at[b, pl.ds(0, R)],
                sem_idx.at[b])

        def gather_copies(b):
            cs = []
            for k in range(NG):
                cs.append(pltpu.make_async_copy(
                    table_hbm.at[idx_v.at[b, pl.ds(128 * k, 128)]],
                    rows_v.at[b, pl.ds(128 * k, 128)], sem_g.at[b]))
            cs.append(pltpu.make_async_copy(
                table_hbm.at[idx_v.at[b, pl.ds(128 * NG, TAIL)]],
                rows_v.at[b, pl.ds(128 * NG, TAIL)], sem_g.at[b]))
            return cs

        def out_copies(c, b):
            return [
                pltpu.make_async_copy(
                    rows_v.at[b, pl.ds(S * k, S)],
                    out_hbm.at[wseq0 + SPS * c + k], sem_out.at[b])
                for k in range(SPS)
            ]

        def start_all(copies):
            for cp in copies:
                cp.start()

        def wait_all(copies):
            for cp in copies:
                cp.wait()

        idx_copy(0, 0).start()
        idx_copy(1, 1).start()
        idx_copy(0, 0).wait()
        start_all(gather_copies(0))

        def group_body(g, carry):
            for b in range(2):
                c = 2 * g + b
                nb = 1 - b
                wait_all(gather_copies(b))

                @pl.when(c + 2 < n_slots)
                def _():
                    idx_copy(c + 2, b).start()

                @pl.when(c + 1 < n_slots)
                def _():
                    idx_copy(0, nb).wait()

                    @pl.when(c + 1 >= 2)
                    def _():
                        wait_all(out_copies(0, nb))

                    start_all(gather_copies(nb))

                def row_body(i, rcarry):
                    for k in range(SPS):
                        for j in range(D // L):
                            plsc.addupdate(
                                rows_v.at[b, S * k + i, pl.ds(j * L, L)],
                                pos_v[i, pl.ds(j * L, L)])
                    return rcarry

                lax.fori_loop(0, S, row_body, 0)

                start_all(out_copies(c, b))
            return carry

        lax.fori_loop(0, n_groups, group_body, 0)

        wait_all(out_copies(0, 0))
        wait_all(out_copies(0, 1))

    return emb_pos_kernel(seq_emb_weight, x.reshape(N), pos)
